# baseline (speedup 1.0000x reference)
import jax
import jax.numpy as jnp
from jax.experimental import pallas as pl
from jax.experimental.pallas import tpu as pltpu


_BLOCK_ROWS = 128


def _fill_body(mask_ref, x_ref, o_ref):
    i = pl.program_id(0)
    m = mask_ref[pl.ds(i * _BLOCK_ROWS, _BLOCK_ROWS), :]
    keep = 1.0 - m.astype(jnp.float32)
    o_ref[...] = x_ref[...] * keep[:, :, None]


def kernel(tensor, mask):
    n, s, d = tensor.shape
    b = _BLOCK_ROWS
    m8 = mask.view(jnp.uint8)
    return pl.pallas_call(
        _fill_body,
        grid=(n // b,),
        in_specs=[
            pl.BlockSpec((n, s), lambda i: (0, 0)),
            pl.BlockSpec((b, s, d), lambda i: (i, 0, 0)),
        ],
        out_specs=pl.BlockSpec((b, s, d), lambda i: (i, 0, 0)),
        out_shape=jax.ShapeDtypeStruct((n, s, d), tensor.dtype),
        compiler_params=pltpu.CompilerParams(
            dimension_semantics=("parallel",),
        ),
    )(m8, tensor)
